# baseline (speedup 1.0000x reference)
import jax
import jax.numpy as jnp
from jax import lax
from jax.experimental import pallas as pl
from jax.experimental.pallas import tpu as pltpu

T = 512
H = 1024
I = 512
E = 16
EPG = 4
G = E // EPG


def _moe_body(x_ref, gw_ref, gb_ref, w1_ref, w2_ref, w3_ref, out_ref,
              comb_ref):
    g = pl.program_id(0)

    @pl.when(g == 0)
    def _router():
        x = x_ref[...]
        logits = lax.dot_general(
            x, gw_ref[...], (((1,), (0,)), ((), ())),
            preferred_element_type=jnp.float32) + gb_ref[...]
        z = logits - jnp.max(logits, axis=1, keepdims=True)
        ez = jnp.exp(z)
        rw = ez / jnp.sum(ez, axis=1, keepdims=True)
        lane = lax.broadcasted_iota(jnp.int32, (T, E), 1)
        m1 = jnp.max(rw, axis=1, keepdims=True)
        e0 = jnp.min(jnp.where(rw == m1, lane, E), axis=1, keepdims=True)
        oh0 = (lane == e0)
        rwx = jnp.where(oh0, -jnp.inf, rw)
        m2 = jnp.max(rwx, axis=1, keepdims=True)
        e1 = jnp.min(jnp.where(rwx == m2, lane, E), axis=1, keepdims=True)
        oh1 = (lane == e1)
        s12 = m1 + m2
        comb_ref[...] = (jnp.where(oh0, m1 / s12, 0.0)
                         + jnp.where(oh1, m2 / s12, 0.0))
        out_ref[...] = jnp.zeros_like(out_ref)

    x = x_ref[...]
    ei = lax.broadcasted_iota(jnp.int32, (E, EPG), 0)
    ec = lax.broadcasted_iota(jnp.int32, (E, EPG), 1)
    sel = (ei == g * EPG + ec).astype(jnp.float32)
    ce2 = lax.dot_general(comb_ref[...], sel, (((1,), (0,)), ((), ())),
                          preferred_element_type=jnp.float32)
    ys = []
    for sub in range(EPG):
        h1 = lax.dot_general(x, w1_ref[sub], (((1,), (1,)), ((), ())),
                             preferred_element_type=jnp.float32)
        h3 = lax.dot_general(x, w3_ref[sub], (((1,), (1,)), ((), ())),
                             preferred_element_type=jnp.float32)
        hh = h1 * jax.nn.sigmoid(h1) * h3
        hc = ce2[:, sub:sub + 1] * hh
        ys.append(lax.dot_general(hc, w2_ref[sub], (((1,), (1,)), ((), ())),
                                  preferred_element_type=jnp.float32))
    acc = ys[0]
    for yv in ys[1:]:
        acc = acc + yv
    out_ref[...] += acc


@jax.jit
def kernel(x, gate_w, gate_b, w1, w2, w3):
    gb2 = gate_b.reshape(1, E)
    return pl.pallas_call(
        _moe_body,
        grid=(G,),
        in_specs=[
            pl.BlockSpec((T, H), lambda g: (0, 0)),
            pl.BlockSpec((H, E), lambda g: (0, 0)),
            pl.BlockSpec((1, E), lambda g: (0, 0)),
            pl.BlockSpec((EPG, I, H), lambda g: (g, 0, 0)),
            pl.BlockSpec((EPG, H, I), lambda g: (g, 0, 0)),
            pl.BlockSpec((EPG, I, H), lambda g: (g, 0, 0)),
        ],
        out_specs=pl.BlockSpec((T, H), lambda g: (0, 0)),
        out_shape=jax.ShapeDtypeStruct((T, H), jnp.float32),
        scratch_shapes=[pltpu.VMEM((T, E), jnp.float32)],
        compiler_params=pltpu.CompilerParams(
            dimension_semantics=("arbitrary",)),
    )(x, gate_w, gb2, w1, w2, w3)


# speedup vs baseline: 1.0311x; 1.0311x over previous
import jax
import jax.numpy as jnp
from jax import lax
from jax.experimental import pallas as pl
from jax.experimental.pallas import tpu as pltpu

T = 512
H = 1024
I = 512
E = 16
EPG = 2
G = E // EPG


def _moe_body(x_ref, gw_ref, gb_ref, w1_ref, w2_ref, w3_ref, out_ref,
              comb_ref):
    g = pl.program_id(0)

    @pl.when(g == 0)
    def _router():
        x = x_ref[...]
        logits = lax.dot_general(
            x, gw_ref[...], (((1,), (0,)), ((), ())),
            preferred_element_type=jnp.float32) + gb_ref[...]
        z = logits - jnp.max(logits, axis=1, keepdims=True)
        ez = jnp.exp(z)
        rw = ez / jnp.sum(ez, axis=1, keepdims=True)
        lane = lax.broadcasted_iota(jnp.int32, (T, E), 1)
        m1 = jnp.max(rw, axis=1, keepdims=True)
        e0 = jnp.min(jnp.where(rw == m1, lane, E), axis=1, keepdims=True)
        oh0 = (lane == e0)
        rwx = jnp.where(oh0, -jnp.inf, rw)
        m2 = jnp.max(rwx, axis=1, keepdims=True)
        e1 = jnp.min(jnp.where(rwx == m2, lane, E), axis=1, keepdims=True)
        oh1 = (lane == e1)
        s12 = m1 + m2
        comb_ref[...] = (jnp.where(oh0, m1 / s12, 0.0)
                         + jnp.where(oh1, m2 / s12, 0.0))
        out_ref[...] = jnp.zeros_like(out_ref)

    x = x_ref[...]
    ei = lax.broadcasted_iota(jnp.int32, (E, EPG), 0)
    ec = lax.broadcasted_iota(jnp.int32, (E, EPG), 1)
    sel = (ei == g * EPG + ec).astype(jnp.float32)
    ce2 = lax.dot_general(comb_ref[...], sel, (((1,), (0,)), ((), ())),
                          preferred_element_type=jnp.float32)
    ys = []
    for sub in range(EPG):
        h1 = lax.dot_general(x, w1_ref[sub], (((1,), (1,)), ((), ())),
                             preferred_element_type=jnp.float32)
        h3 = lax.dot_general(x, w3_ref[sub], (((1,), (1,)), ((), ())),
                             preferred_element_type=jnp.float32)
        hh = h1 * jax.nn.sigmoid(h1) * h3
        hc = ce2[:, sub:sub + 1] * hh
        ys.append(lax.dot_general(hc, w2_ref[sub], (((1,), (1,)), ((), ())),
                                  preferred_element_type=jnp.float32))
    out_ref[...] += ys[0] + ys[1]


@jax.jit
def kernel(x, gate_w, gate_b, w1, w2, w3):
    gb2 = gate_b.reshape(1, E)
    return pl.pallas_call(
        _moe_body,
        grid=(G,),
        in_specs=[
            pl.BlockSpec((T, H), lambda g: (0, 0)),
            pl.BlockSpec((H, E), lambda g: (0, 0)),
            pl.BlockSpec((1, E), lambda g: (0, 0)),
            pl.BlockSpec((EPG, I, H), lambda g: (g, 0, 0)),
            pl.BlockSpec((EPG, H, I), lambda g: (g, 0, 0)),
            pl.BlockSpec((EPG, I, H), lambda g: (g, 0, 0)),
        ],
        out_specs=pl.BlockSpec((T, H), lambda g: (0, 0)),
        out_shape=jax.ShapeDtypeStruct((T, H), jnp.float32),
        scratch_shapes=[pltpu.VMEM((T, E), jnp.float32)],
        compiler_params=pltpu.CompilerParams(
            dimension_semantics=("arbitrary",)),
    )(x, gate_w, gb2, w1, w2, w3)
